# baseline (speedup 1.0000x reference)
import jax
import jax.numpy as jnp
from jax import lax
from jax.experimental import pallas as pl
from jax.experimental.pallas import tpu as pltpu
from jax.experimental.pallas import tpu_sc as plsc

N = 100000
H = 128
S = 64
NW = 32
ROWS_PW = 3200
CHUNK = 80
NCH_FULL = ROWS_PW // CHUNK
NCH_LAST = (N - (NW - 1) * ROWS_PW) // CHUNK
NPAD = NW * ROWS_PW
NGC = CHUNK // 16
NEG = float("-inf")

_MESH = dict(core_axis_name="c", subcore_axis_name="s")


def _wid():
    return lax.axis_index("c") * 16 + lax.axis_index("s")


def _nchunks(wid):
    return jnp.where(wid == NW - 1, NCH_LAST, NCH_FULL)


def _xcopy(x_hbm, base, ci, buf, sem):
    src = x_hbm.at[pl.ds(base + ci * CHUNK, CHUNK), :]
    return pltpu.make_async_copy(src, buf, sem)


def _main_body(x_hbm, b_hbm, w_hbm, mpart_hbm, spart_hbm, apart_hbm,
               xb0, xb1, bbuf, wv, mloc, sloc, abuf, sem):
    wid = _wid()
    base = wid * ROWS_PW
    nch = _nchunks(wid)
    iota = lax.iota(jnp.int32, 16)
    pltpu.sync_copy(w_hbm, wv)
    Q = ROWS_PW // 4
    for q in range(4):
        @pl.when(jnp.logical_or(q == 0, wid < NW - 1))
        def _():
            pltpu.sync_copy(b_hbm.at[pl.ds(base + q * Q, Q)],
                            bbuf.at[pl.ds(q * Q, Q)])
    _xcopy(x_hbm, base, 0, xb0, sem).start()
    wvec = [wv[pl.ds(16 * t, 16)] for t in range(H // 16)]
    for t in range(S // 16):
        mloc[pl.ds(16 * t, 16)] = jnp.full((16,), NEG, jnp.float32)
        sloc[pl.ds(16 * t, 16)] = jnp.zeros((16,), jnp.float32)

    def zrow(r, c2):
        for j in range(H // 16):
            abuf[r, pl.ds(16 * j, 16)] = jnp.zeros((16,), jnp.float32)
        return c2
    lax.fori_loop(0, S, zrow, 0)

    def compute(ci, xb):
        def group_body(gi, c2):
            rb = gi * 16
            b16 = bbuf[pl.ds(ci * CHUNK + rb, 16)]
            gp = [jnp.zeros((16,), jnp.float32) for _ in range(4)]
            for r2 in range(16):
                alo = xb[rb + r2, pl.ds(0, 16)] * wvec[0]
                ahi = xb[rb + r2, pl.ds(64, 16)] * wvec[4]
                for j in range(1, 4):
                    alo = alo + xb[rb + r2, pl.ds(16 * j, 16)] * wvec[j]
                    ahi = ahi + \
                        xb[rb + r2, pl.ds(64 + 16 * j, 16)] * wvec[4 + j]
                gp[r2 % 4] = jnp.where(iota == r2,
                                       jnp.sum(alo + ahi), gp[r2 % 4])
            g16 = (gp[0] + gp[1]) + (gp[2] + gp[3])

            def fast():
                b = b16[0]
                t = b // 16
                lane = b - t * 16
                gm = jnp.max(g16)
                mo = plsc.load_gather(mloc, [b16])
                mn = jnp.maximum(mo, gm)
                c = jnp.exp(mo - mn)
                w16 = jnp.exp(g16 - mn)
                mv = mloc[pl.ds(t * 16, 16)]
                mloc[pl.ds(t * 16, 16)] = jnp.where(iota == lane, mn, mv)
                sw = jnp.sum(w16)
                sv = sloc[pl.ds(t * 16, 16)]
                sloc[pl.ds(t * 16, 16)] = jnp.where(
                    iota == lane, sv * c + sw, sv)
                acc = [w16[0] * xb[rb, pl.ds(16 * j, 16)]
                       for j in range(H // 16)]
                for r2 in range(1, 16):
                    for j in range(H // 16):
                        acc[j] = acc[j] + \
                            w16[r2] * xb[rb + r2, pl.ds(16 * j, 16)]
                for j in range(H // 16):
                    sl = pl.ds(16 * j, 16)
                    abuf[b, sl] = abuf[b, sl] * c + acc[j]

            def slow():
                for r2 in range(16):
                    b = b16[r2]
                    t = b // 16
                    lane = b - t * 16
                    g = g16[r2]
                    mo = plsc.load_gather(mloc, [iota * 0 + b])
                    mn = jnp.maximum(mo, g)
                    c = jnp.exp(mo - mn)
                    w = jnp.exp(g - mn)
                    mv = mloc[pl.ds(t * 16, 16)]
                    mloc[pl.ds(t * 16, 16)] = jnp.where(
                        iota == lane, mn, mv)
                    sv = sloc[pl.ds(t * 16, 16)]
                    sloc[pl.ds(t * 16, 16)] = jnp.where(
                        iota == lane, sv * c + w, sv)
                    for j in range(H // 16):
                        sl = pl.ds(16 * j, 16)
                        abuf[b, sl] = abuf[b, sl] * c + \
                            w * xb[rb + r2, sl]

            lax.cond(b16[0] == b16[15], fast, slow)
            return c2

        lax.fori_loop(0, NGC, group_body, 0, unroll=2)

    def pair_body(p, carry):
        ci0 = 2 * p
        ci1 = 2 * p + 1
        _xcopy(x_hbm, base, ci0, xb0, sem).wait()
        _xcopy(x_hbm, base, ci1, xb1, sem).start()
        compute(ci0, xb0)
        _xcopy(x_hbm, base, ci1, xb1, sem).wait()

        @pl.when(ci1 + 1 < nch)
        def _():
            _xcopy(x_hbm, base, ci1 + 1, xb0, sem).start()

        compute(ci1, xb1)
        return carry

    lax.fori_loop(0, nch // 2, pair_body, 0)
    pltpu.sync_copy(mloc, mpart_hbm.at[pl.ds(wid * S, S)])
    pltpu.sync_copy(sloc, spart_hbm.at[pl.ds(wid * S, S)])
    pltpu.sync_copy(abuf, apart_hbm.at[pl.ds(wid * S, S), :])


def _merge_body(mpart_hbm, spart_hbm, a2_hbm, out_hbm,
                mp, sp, idxbuf, rows, ob, sem):
    wid = _wid()
    iota = lax.iota(jnp.int32, 16)
    pltpu.sync_copy(mpart_hbm, mp)
    pltpu.sync_copy(spart_hbm, sp)
    for k in range(S // NW):
        seg = wid * (S // NW) + k
        ia = iota * S + seg
        ib = ia + 16 * S
        m16a = plsc.load_gather(mp, [ia])
        m16b = plsc.load_gather(mp, [ib])
        mg = jnp.maximum(jnp.max(jnp.maximum(m16a, m16b)),
                         jnp.float32(-1e38))
        f16a = jnp.exp(m16a - mg)
        f16b = jnp.exp(m16b - mg)
        s16a = plsc.load_gather(sp, [ia])
        s16b = plsc.load_gather(sp, [ib])
        sden = jnp.sum(s16a * f16a) + jnp.sum(s16b * f16b)
        inv = 1.0 / (jnp.zeros((16,), jnp.float32) + sden + 1e-16)
        idxbuf[pl.ds(0, 16)] = ia
        idxbuf[pl.ds(16, 16)] = ib
        pltpu.async_copy(a2_hbm.at[idxbuf], rows, sem).wait()
        for j in range(H // 16):
            sl = pl.ds(16 * j, 16)
            o = f16a[0] * rows[0, sl]
            for w2 in range(1, NW):
                f = f16a[w2] if w2 < 16 else f16b[w2 - 16]
                o = o + f * rows[w2, sl]
            ob[sl] = o * inv
        pltpu.sync_copy(ob, out_hbm.at[seg])


def _make_kernels():
    mesh = plsc.VectorSubcoreMesh(**_MESH)
    cp = pltpu.CompilerParams(needs_layout_passes=False)
    pmain = pl.kernel(
        _main_body,
        out_type=[jax.ShapeDtypeStruct((NW * S,), jnp.float32),
                  jax.ShapeDtypeStruct((NW * S,), jnp.float32),
                  jax.ShapeDtypeStruct((NW * S, H), jnp.float32)],
        mesh=mesh,
        scratch_types=[pltpu.VMEM((CHUNK, H), jnp.float32),
                       pltpu.VMEM((CHUNK, H), jnp.float32),
                       pltpu.VMEM((ROWS_PW,), jnp.int32),
                       pltpu.VMEM((H,), jnp.float32),
                       pltpu.VMEM((S,), jnp.float32),
                       pltpu.VMEM((S,), jnp.float32),
                       pltpu.VMEM((S, H), jnp.float32),
                       pltpu.SemaphoreType.DMA],
        compiler_params=cp,
    )
    pmerge = pl.kernel(
        _merge_body,
        out_type=jax.ShapeDtypeStruct((S, H), jnp.float32),
        mesh=mesh,
        scratch_types=[pltpu.VMEM((NW * S,), jnp.float32),
                       pltpu.VMEM((NW * S,), jnp.float32),
                       pltpu.VMEM((NW,), jnp.int32),
                       pltpu.VMEM((NW, H), jnp.float32),
                       pltpu.VMEM((H,), jnp.float32),
                       pltpu.SemaphoreType.DMA],
        compiler_params=cp,
    )
    return pmain, pmerge


_PMAIN, _PMERGE = _make_kernels()


@jax.jit
def kernel(x, batch, W):
    w = W.reshape((H,))
    mpart, spart, apart = _PMAIN(x, batch, w)
    return _PMERGE(mpart, spart, apart)


# speedup vs baseline: 1.7122x; 1.7122x over previous
import jax
import jax.numpy as jnp
from jax import lax
from jax.experimental import pallas as pl
from jax.experimental.pallas import tpu as pltpu
from jax.experimental.pallas import tpu_sc as plsc

N = 100000
H = 128
S = 64
NW = 32
ROWS_PW = 3200
CHUNK = 80
NCH_FULL = ROWS_PW // CHUNK
NCH_LAST = (N - (NW - 1) * ROWS_PW) // CHUNK
NPAD = NW * ROWS_PW
NGC = CHUNK // 16
NEG = float("-inf")

_MESH = dict(core_axis_name="c", subcore_axis_name="s")


def _wid():
    return lax.axis_index("c") * 16 + lax.axis_index("s")


def _nchunks(wid):
    return jnp.where(wid == NW - 1, NCH_LAST, NCH_FULL)


def _xcopy(x_hbm, base, ci, buf, sem):
    src = x_hbm.at[pl.ds(base + ci * CHUNK, CHUNK), :]
    return pltpu.make_async_copy(src, buf, sem)


def _main_body(x_hbm, b_hbm, w_hbm, mpart_hbm, spart_hbm, apart_hbm,
               xb0, xb1, bbuf, wv, mloc, sloc, abuf, sem):
    wid = _wid()
    base = wid * ROWS_PW
    nch = _nchunks(wid)
    iota = lax.iota(jnp.int32, 16)
    pltpu.sync_copy(w_hbm, wv)
    Q = ROWS_PW // 4
    for q in range(4):
        @pl.when(jnp.logical_or(q == 0, wid < NW - 1))
        def _():
            pltpu.sync_copy(b_hbm.at[pl.ds(base + q * Q, Q)],
                            bbuf.at[pl.ds(q * Q, Q)])
    _xcopy(x_hbm, base, 0, xb0, sem).start()
    wvec = [wv[pl.ds(16 * t, 16)] for t in range(H // 16)]
    for t in range(S // 16):
        mloc[pl.ds(16 * t, 16)] = jnp.full((16,), NEG, jnp.float32)
        sloc[pl.ds(16 * t, 16)] = jnp.zeros((16,), jnp.float32)

    def zrow(r, c2):
        for j in range(H // 16):
            abuf[r, pl.ds(16 * j, 16)] = jnp.zeros((16,), jnp.float32)
        return c2
    lax.fori_loop(0, S, zrow, 0)

    def compute(ci, xb):
        def group_body(gi, c2):
            rb = gi * 16
            b16 = bbuf[pl.ds(ci * CHUNK + rb, 16)]
            gp = [jnp.zeros((16,), jnp.float32) for _ in range(4)]
            for r2 in range(16):
                alo = xb[rb + r2, pl.ds(0, 16)] * wvec[0]
                ahi = xb[rb + r2, pl.ds(64, 16)] * wvec[4]
                for j in range(1, 4):
                    alo = alo + xb[rb + r2, pl.ds(16 * j, 16)] * wvec[j]
                    ahi = ahi + \
                        xb[rb + r2, pl.ds(64 + 16 * j, 16)] * wvec[4 + j]
                gp[r2 % 4] = jnp.where(iota == r2,
                                       jnp.sum(alo + ahi), gp[r2 % 4])
            g16 = (gp[0] + gp[1]) + (gp[2] + gp[3])

            def fast():
                b = b16[0]
                t = b // 16
                lane = b - t * 16
                gm = jnp.max(g16)
                mo = plsc.load_gather(mloc, [b16])
                mn = jnp.maximum(mo, gm)
                c = jnp.exp(mo - mn)
                w16 = jnp.exp(g16 - mn)
                mv = mloc[pl.ds(t * 16, 16)]
                mloc[pl.ds(t * 16, 16)] = jnp.where(iota == lane, mn, mv)
                sw = jnp.sum(w16)
                sv = sloc[pl.ds(t * 16, 16)]
                sloc[pl.ds(t * 16, 16)] = jnp.where(
                    iota == lane, sv * c + sw, sv)
                acc = [w16[0] * xb[rb, pl.ds(16 * j, 16)]
                       for j in range(H // 16)]
                for r2 in range(1, 16):
                    for j in range(H // 16):
                        acc[j] = acc[j] + \
                            w16[r2] * xb[rb + r2, pl.ds(16 * j, 16)]
                for j in range(H // 16):
                    sl = pl.ds(16 * j, 16)
                    abuf[b, sl] = abuf[b, sl] * c + acc[j]

            def slow():
                for r2 in range(16):
                    b = b16[r2]
                    t = b // 16
                    lane = b - t * 16
                    g = g16[r2]
                    mo = plsc.load_gather(mloc, [iota * 0 + b])
                    mn = jnp.maximum(mo, g)
                    c = jnp.exp(mo - mn)
                    w = jnp.exp(g - mn)
                    mv = mloc[pl.ds(t * 16, 16)]
                    mloc[pl.ds(t * 16, 16)] = jnp.where(
                        iota == lane, mn, mv)
                    sv = sloc[pl.ds(t * 16, 16)]
                    sloc[pl.ds(t * 16, 16)] = jnp.where(
                        iota == lane, sv * c + w, sv)
                    for j in range(H // 16):
                        sl = pl.ds(16 * j, 16)
                        abuf[b, sl] = abuf[b, sl] * c + \
                            w * xb[rb + r2, sl]

            lax.cond(b16[0] == b16[15], fast, slow)
            return c2

        lax.fori_loop(0, NGC, group_body, 0)

    def pair_body(p, carry):
        ci0 = 2 * p
        ci1 = 2 * p + 1
        _xcopy(x_hbm, base, ci0, xb0, sem).wait()
        _xcopy(x_hbm, base, ci1, xb1, sem).start()
        compute(ci0, xb0)
        _xcopy(x_hbm, base, ci1, xb1, sem).wait()

        @pl.when(ci1 + 1 < nch)
        def _():
            _xcopy(x_hbm, base, ci1 + 1, xb0, sem).start()

        compute(ci1, xb1)
        return carry

    lax.fori_loop(0, nch // 2, pair_body, 0)
    pltpu.sync_copy(mloc, mpart_hbm.at[pl.ds(wid * S, S)])
    pltpu.sync_copy(sloc, spart_hbm.at[pl.ds(wid * S, S)])
    pltpu.sync_copy(abuf, apart_hbm.at[pl.ds(wid * S, S), :])


def _merge_body(mpart_hbm, spart_hbm, a2_hbm, out_hbm,
                mp, sp, idxbuf, rows, ob, sem):
    wid = _wid()
    iota = lax.iota(jnp.int32, 16)
    pltpu.sync_copy(mpart_hbm, mp)
    pltpu.sync_copy(spart_hbm, sp)
    for k in range(S // NW):
        seg = wid * (S // NW) + k
        ia = iota * S + seg
        ib = ia + 16 * S
        m16a = plsc.load_gather(mp, [ia])
        m16b = plsc.load_gather(mp, [ib])
        mg = jnp.maximum(jnp.max(jnp.maximum(m16a, m16b)),
                         jnp.float32(-1e38))
        f16a = jnp.exp(m16a - mg)
        f16b = jnp.exp(m16b - mg)
        s16a = plsc.load_gather(sp, [ia])
        s16b = plsc.load_gather(sp, [ib])
        sden = jnp.sum(s16a * f16a) + jnp.sum(s16b * f16b)
        inv = 1.0 / (jnp.zeros((16,), jnp.float32) + sden + 1e-16)
        idxbuf[pl.ds(0, 16)] = ia
        idxbuf[pl.ds(16, 16)] = ib
        pltpu.async_copy(a2_hbm.at[idxbuf], rows, sem).wait()
        for j in range(H // 16):
            sl = pl.ds(16 * j, 16)
            o = f16a[0] * rows[0, sl]
            for w2 in range(1, NW):
                f = f16a[w2] if w2 < 16 else f16b[w2 - 16]
                o = o + f * rows[w2, sl]
            ob[sl] = o * inv
        pltpu.sync_copy(ob, out_hbm.at[seg])


def _make_kernels():
    mesh = plsc.VectorSubcoreMesh(**_MESH)
    cp = pltpu.CompilerParams(needs_layout_passes=False)
    pmain = pl.kernel(
        _main_body,
        out_type=[jax.ShapeDtypeStruct((NW * S,), jnp.float32),
                  jax.ShapeDtypeStruct((NW * S,), jnp.float32),
                  jax.ShapeDtypeStruct((NW * S, H), jnp.float32)],
        mesh=mesh,
        scratch_types=[pltpu.VMEM((CHUNK, H), jnp.float32),
                       pltpu.VMEM((CHUNK, H), jnp.float32),
                       pltpu.VMEM((ROWS_PW,), jnp.int32),
                       pltpu.VMEM((H,), jnp.float32),
                       pltpu.VMEM((S,), jnp.float32),
                       pltpu.VMEM((S,), jnp.float32),
                       pltpu.VMEM((S, H), jnp.float32),
                       pltpu.SemaphoreType.DMA],
        compiler_params=cp,
    )
    pmerge = pl.kernel(
        _merge_body,
        out_type=jax.ShapeDtypeStruct((S, H), jnp.float32),
        mesh=mesh,
        scratch_types=[pltpu.VMEM((NW * S,), jnp.float32),
                       pltpu.VMEM((NW * S,), jnp.float32),
                       pltpu.VMEM((NW,), jnp.int32),
                       pltpu.VMEM((NW, H), jnp.float32),
                       pltpu.VMEM((H,), jnp.float32),
                       pltpu.SemaphoreType.DMA],
        compiler_params=cp,
    )
    return pmain, pmerge


_PMAIN, _PMERGE = _make_kernels()


@jax.jit
def kernel(x, batch, W):
    w = W.reshape((H,))
    mpart, spart, apart = _PMAIN(x, batch, w)
    return _PMERGE(mpart, spart, apart)
